# baseline (speedup 1.0000x reference)
import jax
import jax.numpy as jnp
from jax import lax
from jax.experimental import pallas as pl
from jax.experimental.pallas import tpu as pltpu
from jax.experimental.pallas import tpu_sc as plsc

VOCAB = 100000
EMBED = 64
HIDDEN = 128
BATCH = 1024
SEQ = 50

NUM_WORKERS = 32
ROWS_PER_W = BATCH * SEQ // NUM_WORKERS
CHUNK = 64
NCHUNK = ROWS_PER_W // CHUNK
USPLIT = 13

VHALF = 50048
TW = 2944
TGRID = VHALF // TW


def _transpose_body(lo_ref, hi_ref, out_ref):
    out_ref[...] = jnp.concatenate([lo_ref[...].T, hi_ref[...].T], axis=1)


def _tc_transpose(view, interpret=False):
    return pl.pallas_call(
        _transpose_body,
        grid=(TGRID,),
        in_specs=[
            pl.BlockSpec((EMBED, TW), lambda j: (0, j)),
            pl.BlockSpec((EMBED, TW), lambda j: (0, j + TGRID)),
        ],
        out_specs=pl.BlockSpec((TW, 2 * EMBED), lambda j: (j, 0)),
        out_shape=jax.ShapeDtypeStruct((VHALF, 2 * EMBED), jnp.float32),
        interpret=interpret,
    )(view, view)


def _idx_body(leftT_hbm, idx_hbm, left_v, idx_v):
    c = lax.axis_index("c")
    s = lax.axis_index("s")
    wid = s * 2 + c
    base = wid * ROWS_PER_W
    pltpu.sync_copy(leftT_hbm, left_v)
    for ch in range(NCHUNK):
        for q in range(CHUNK // 16):
            j = base + ch * CHUNK + q * 16 + lax.iota(jnp.int32, 16)
            k = j >> 1
            b = k & (BATCH - 1)
            u = k >> 10
            t = (u << 1) | (j & 1)
            vals = plsc.load_gather(left_v, [t, b])
            m = jnp.where(vals < VHALF, vals * 2, vals * 2 - (2 * VHALF - 1))
            idx_v[ch, pl.ds(q * 16, 16)] = m
    pltpu.sync_copy(idx_v, idx_hbm.at[pl.ds(wid * NCHUNK, NCHUNK)])


def _sc_idx(left):
    mesh = plsc.VectorSubcoreMesh(core_axis_name="c", subcore_axis_name="s")
    run = pl.kernel(
        _idx_body,
        mesh=mesh,
        out_type=jax.ShapeDtypeStruct((NUM_WORKERS * NCHUNK, CHUNK), jnp.int32),
        scratch_types=[
            pltpu.VMEM((SEQ, BATCH), jnp.int32),
            pltpu.VMEM((NCHUNK, CHUNK), jnp.int32),
        ],
        compiler_params=pltpu.CompilerParams(
            use_tc_tiling_on_sc=False, needs_layout_passes=False
        ),
    )
    return run(left)


def _sc_gather_part(table2, idxs, start, nch):
    rpw = nch * CHUNK
    chunk_row0 = start // CHUNK

    def body(table_hbm, idxs_hbm, out_hbm, idx_v, rows_v, sem):
        c = lax.axis_index("c")
        s = lax.axis_index("s")
        wid = s * 2 + c
        pltpu.sync_copy(
            idxs_hbm.at[pl.ds(chunk_row0 + wid * nch, nch)], idx_v
        )
        copies = []
        for ch in range(nch):
            copies.append(
                pltpu.async_copy(
                    table_hbm.at[idx_v.at[ch]],
                    rows_v.at[pl.ds(ch * CHUNK, CHUNK)],
                    sem,
                )
            )
        for cp in copies:
            cp.wait()
        pltpu.sync_copy(rows_v, out_hbm.at[pl.ds(wid * rpw, rpw)])

    mesh = plsc.VectorSubcoreMesh(core_axis_name="c", subcore_axis_name="s")
    run = pl.kernel(
        body,
        mesh=mesh,
        out_type=jax.ShapeDtypeStruct((NUM_WORKERS * rpw, EMBED), jnp.float32),
        scratch_types=[
            pltpu.VMEM((nch, CHUNK), jnp.int32),
            pltpu.VMEM((rpw, EMBED), jnp.float32),
            pltpu.SemaphoreType.DMA,
        ],
        compiler_params=pltpu.CompilerParams(
            use_tc_tiling_on_sc=False, needs_layout_passes=False
        ),
    )
    return run(table2, idxs)


def _tc_lstm_chunk(len2, xs, wcat_e, wcat_o, bias, h0, c0, u_base, n,
                   interpret=False):
    def body(len_ref, xs_ref, we_ref, wo_ref, b_ref, h0_ref, c0_ref,
             h_out, c_out, cat_scr, c_scr):
        u = pl.program_id(0)

        @pl.when(u == 0)
        def _init():
            cat_scr[:, 2 * EMBED:] = h0_ref[...]
            c_scr[...] = c0_ref[...]

        cat_scr[:, 0:2 * EMBED] = xs_ref[0]
        for p in range(2):
            h = cat_scr[:, 2 * EMBED:]
            c = c_scr[...]
            w_ref = we_ref if p == 0 else wo_ref
            gates = (
                jnp.dot(
                    cat_scr[...], w_ref[...],
                    preferred_element_type=jnp.float32,
                )
                + b_ref[...]
            )
            tg = jnp.tanh(gates)
            i_g = tg[:, 0 * HIDDEN:1 * HIDDEN] * 0.5 + 0.5
            f_g = tg[:, 1 * HIDDEN:2 * HIDDEN] * 0.5 + 0.5
            g_g = tg[:, 2 * HIDDEN:3 * HIDDEN]
            o_g = tg[:, 3 * HIDDEN:4 * HIDDEN] * 0.5 + 0.5
            c_new = f_g * c + i_g * g_g
            h_new = o_g * jnp.tanh(c_new)
            m = len_ref[...] > (2 * (u_base + u) + p)
            cat_scr[:, 2 * EMBED:] = jnp.where(m, h_new, h)
            c_scr[...] = jnp.where(m, c_new, c)

        @pl.when(u == n - 1)
        def _fin():
            h_out[...] = cat_scr[:, 2 * EMBED:]
            c_out[...] = c_scr[...]

    return pl.pallas_call(
        body,
        grid=(n,),
        in_specs=[
            pl.BlockSpec((BATCH, 1), lambda u: (0, 0)),
            pl.BlockSpec((1, BATCH, 2 * EMBED), lambda u: (u, 0, 0)),
            pl.BlockSpec((2 * EMBED + HIDDEN, 4 * HIDDEN), lambda u: (0, 0)),
            pl.BlockSpec((2 * EMBED + HIDDEN, 4 * HIDDEN), lambda u: (0, 0)),
            pl.BlockSpec((1, 4 * HIDDEN), lambda u: (0, 0)),
            pl.BlockSpec((BATCH, HIDDEN), lambda u: (0, 0)),
            pl.BlockSpec((BATCH, HIDDEN), lambda u: (0, 0)),
        ],
        out_specs=[
            pl.BlockSpec((BATCH, HIDDEN), lambda u: (0, 0)),
            pl.BlockSpec((BATCH, HIDDEN), lambda u: (0, 0)),
        ],
        out_shape=[
            jax.ShapeDtypeStruct((BATCH, HIDDEN), jnp.float32),
            jax.ShapeDtypeStruct((BATCH, HIDDEN), jnp.float32),
        ],
        scratch_shapes=[
            pltpu.VMEM((BATCH, 2 * EMBED + HIDDEN), jnp.float32),
            pltpu.VMEM((BATCH, HIDDEN), jnp.float32),
        ],
        interpret=interpret,
    )(len2, xs, wcat_e, wcat_o, bias, h0, c0)


def kernel(left, left_len, word_emb, W_ih, W_hh, b_ih, b_hh):
    view = word_emb.T
    fused = _tc_transpose(view)
    table2 = fused.reshape(2 * VHALF, EMBED)
    idxs = _sc_idx(left.astype(jnp.int32).T)
    ga = _sc_gather_part(table2, idxs, 0, USPLIT)
    gb = _sc_gather_part(table2, idxs, USPLIT * 2 * BATCH, SEQ // 2 - USPLIT)
    xa = ga.reshape(USPLIT, BATCH, 2 * EMBED)
    xb = gb.reshape(SEQ // 2 - USPLIT, BATCH, 2 * EMBED)
    scale = jnp.concatenate(
        [
            jnp.full((2 * HIDDEN,), 0.5, jnp.float32),
            jnp.ones((HIDDEN,), jnp.float32),
            jnp.full((HIDDEN,), 0.5, jnp.float32),
        ]
    )
    wx = W_ih.T * scale[None, :]
    wh = W_hh.T * scale[None, :]
    z = jnp.zeros((EMBED, 4 * HIDDEN), jnp.float32)
    wcat_e = jnp.concatenate([wx, z, wh])
    wcat_o = jnp.concatenate([z, wx, wh])
    bias = ((b_ih + b_hh) * scale).reshape(1, 4 * HIDDEN)
    len2 = left_len.reshape(BATCH, 1).astype(jnp.int32)
    z = jnp.zeros((BATCH, HIDDEN), jnp.float32)
    h1, c1 = _tc_lstm_chunk(len2, xa, wcat_e, wcat_o, bias, z, z, 0, USPLIT)
    h2, _ = _tc_lstm_chunk(
        len2, xb, wcat_e, wcat_o, bias, h1, c1, USPLIT, SEQ // 2 - USPLIT
    )
    return h2


# speedup vs baseline: 1.0223x; 1.0223x over previous
import jax
import jax.numpy as jnp
from jax import lax
from jax.experimental import pallas as pl
from jax.experimental.pallas import tpu as pltpu
from jax.experimental.pallas import tpu_sc as plsc

VOCAB = 100000
EMBED = 64
HIDDEN = 128
BATCH = 1024
SEQ = 50

NUM_WORKERS = 32
ROWS_PER_W = BATCH * SEQ // NUM_WORKERS
CHUNK = 80
NCHUNK = ROWS_PER_W // CHUNK

VHALF = 50048
TW = 2944
TGRID = VHALF // TW


def _transpose_body(lo_ref, hi_ref, out_ref):
    out_ref[...] = jnp.concatenate([lo_ref[...].T, hi_ref[...].T], axis=1)


def _tc_transpose(view, interpret=False):
    return pl.pallas_call(
        _transpose_body,
        grid=(TGRID,),
        in_specs=[
            pl.BlockSpec((EMBED, TW), lambda j: (0, j)),
            pl.BlockSpec((EMBED, TW), lambda j: (0, j + TGRID)),
        ],
        out_specs=pl.BlockSpec((TW, 2 * EMBED), lambda j: (j, 0)),
        out_shape=jax.ShapeDtypeStruct((VHALF, 2 * EMBED), jnp.float32),
        interpret=interpret,
    )(view, view)


def _idx_body(leftT_hbm, idx_hbm, left_v, idx_v):
    c = lax.axis_index("c")
    s = lax.axis_index("s")
    wid = s * 2 + c
    base = wid * ROWS_PER_W
    pltpu.sync_copy(leftT_hbm, left_v)
    for ch in range(NCHUNK):
        for q in range(CHUNK // 16):
            j = base + ch * CHUNK + q * 16 + lax.iota(jnp.int32, 16)
            k = j >> 1
            b = k & (BATCH - 1)
            u = k >> 10
            t = (u << 1) | (j & 1)
            vals = plsc.load_gather(left_v, [t, b])
            m = jnp.where(vals < VHALF, vals * 2, vals * 2 - (2 * VHALF - 1))
            idx_v[ch, pl.ds(q * 16, 16)] = m
    pltpu.sync_copy(idx_v, idx_hbm.at[wid])


def _sc_idx(left):
    mesh = plsc.VectorSubcoreMesh(core_axis_name="c", subcore_axis_name="s")
    run = pl.kernel(
        _idx_body,
        mesh=mesh,
        out_type=jax.ShapeDtypeStruct((NUM_WORKERS, NCHUNK, CHUNK), jnp.int32),
        scratch_types=[
            pltpu.VMEM((SEQ, BATCH), jnp.int32),
            pltpu.VMEM((NCHUNK, CHUNK), jnp.int32),
        ],
        compiler_params=pltpu.CompilerParams(
            use_tc_tiling_on_sc=False, needs_layout_passes=False
        ),
    )
    return run(left)


def _gather_body(table_hbm, idxs_hbm, out_hbm, idx_v, rows_v, sem):
    c = lax.axis_index("c")
    s = lax.axis_index("s")
    wid = s * 2 + c
    base = wid * ROWS_PER_W
    pltpu.sync_copy(idxs_hbm.at[wid], idx_v)
    copies = []
    for ch in range(NCHUNK):
        copies.append(
            pltpu.async_copy(
                table_hbm.at[idx_v.at[ch]],
                rows_v.at[pl.ds(ch * CHUNK, CHUNK)],
                sem,
            )
        )
    for cp in copies:
        cp.wait()
    pltpu.sync_copy(rows_v, out_hbm.at[pl.ds(base, ROWS_PER_W)])


def _sc_gather(table2, idxs):
    mesh = plsc.VectorSubcoreMesh(core_axis_name="c", subcore_axis_name="s")
    run = pl.kernel(
        _gather_body,
        mesh=mesh,
        out_type=jax.ShapeDtypeStruct((SEQ * BATCH, EMBED), jnp.float32),
        scratch_types=[
            pltpu.VMEM((NCHUNK, CHUNK), jnp.int32),
            pltpu.VMEM((ROWS_PER_W, EMBED), jnp.float32),
            pltpu.SemaphoreType.DMA,
        ],
        compiler_params=pltpu.CompilerParams(
            use_tc_tiling_on_sc=False, needs_layout_passes=False
        ),
    )
    return run(table2, idxs)


def _sc_gather_pipeline(word_emb, left, interpret=False):
    view = word_emb.T
    fused = _tc_transpose(view, interpret=interpret)
    table2 = fused.reshape(2 * VHALF, EMBED)
    idxs = _sc_idx(left.T)
    return _sc_gather(table2, idxs)


def _lstm_body(len_ref, xs_ref, we_ref, wo_ref, b_ref, out_ref, cat_scr, c_scr):
    u = pl.program_id(0)

    @pl.when(u == 0)
    def _init():
        cat_scr[...] = jnp.zeros_like(cat_scr)
        c_scr[...] = jnp.zeros_like(c_scr)

    cat_scr[:, 0:2 * EMBED] = xs_ref[0]
    for p in range(2):
        h = cat_scr[:, 2 * EMBED:]
        c = c_scr[...]
        w_ref = we_ref if p == 0 else wo_ref
        gates = (
            jnp.dot(
                cat_scr[...], w_ref[...],
                preferred_element_type=jnp.float32,
            )
            + b_ref[...]
        )
        tg = jnp.tanh(gates)
        i_g = tg[:, 0 * HIDDEN:1 * HIDDEN] * 0.5 + 0.5
        f_g = tg[:, 1 * HIDDEN:2 * HIDDEN] * 0.5 + 0.5
        g_g = tg[:, 2 * HIDDEN:3 * HIDDEN]
        o_g = tg[:, 3 * HIDDEN:4 * HIDDEN] * 0.5 + 0.5
        c_new = f_g * c + i_g * g_g
        h_new = o_g * jnp.tanh(c_new)
        m = len_ref[...] > (2 * u + p)
        cat_scr[:, 2 * EMBED:] = jnp.where(m, h_new, h)
        c_scr[...] = jnp.where(m, c_new, c)

    @pl.when(u == SEQ // 2 - 1)
    def _fin():
        out_ref[...] = cat_scr[:, 2 * EMBED:]


def _tc_lstm(len2, xs, wcat_e, wcat_o, bias, interpret=False):
    return pl.pallas_call(
        _lstm_body,
        grid=(SEQ // 2,),
        in_specs=[
            pl.BlockSpec((BATCH, 1), lambda u: (0, 0)),
            pl.BlockSpec((1, BATCH, 2 * EMBED), lambda u: (u, 0, 0)),
            pl.BlockSpec((2 * EMBED + HIDDEN, 4 * HIDDEN), lambda u: (0, 0)),
            pl.BlockSpec((2 * EMBED + HIDDEN, 4 * HIDDEN), lambda u: (0, 0)),
            pl.BlockSpec((1, 4 * HIDDEN), lambda u: (0, 0)),
        ],
        out_specs=pl.BlockSpec((BATCH, HIDDEN), lambda u: (0, 0)),
        out_shape=jax.ShapeDtypeStruct((BATCH, HIDDEN), jnp.float32),
        scratch_shapes=[
            pltpu.VMEM((BATCH, 2 * EMBED + HIDDEN), jnp.float32),
            pltpu.VMEM((BATCH, HIDDEN), jnp.float32),
        ],
        interpret=interpret,
    )(len2, xs, wcat_e, wcat_o, bias)


def kernel(left, left_len, word_emb, W_ih, W_hh, b_ih, b_hh):
    emb_flat = _sc_gather_pipeline(word_emb, left.astype(jnp.int32))
    xs = emb_flat.reshape(SEQ // 2, BATCH, 2 * EMBED)
    scale = jnp.concatenate(
        [
            jnp.full((2 * HIDDEN,), 0.5, jnp.float32),
            jnp.ones((HIDDEN,), jnp.float32),
            jnp.full((HIDDEN,), 0.5, jnp.float32),
        ]
    )
    wx = W_ih.T * scale[None, :]
    wh = W_hh.T * scale[None, :]
    z = jnp.zeros((EMBED, 4 * HIDDEN), jnp.float32)
    wcat_e = jnp.concatenate([wx, z, wh])
    wcat_o = jnp.concatenate([z, wx, wh])
    bias = ((b_ih + b_hh) * scale).reshape(1, 4 * HIDDEN)
    len2 = left_len.reshape(BATCH, 1).astype(jnp.int32)
    return _tc_lstm(len2, xs, wcat_e, wcat_o, bias)
